# baseline (speedup 1.0000x reference)
import jax
import jax.numpy as jnp
from jax import lax
from jax.experimental import pallas as pl
from jax.experimental.pallas import tpu as pltpu
from jax.experimental.pallas import tpu_sc as plsc

R, C = 128, 100000
NC, NS = 2, 16
NW = NC * NS

T_SPLIT = 32000
Q = 1000
Q_LAST0 = T_SPLIT - Q
CKT = 200
NCH = Q // CKT

TC_BW = 4000
TC_BLK0 = T_SPLIT // TC_BW
TC_NBLK = (C - T_SPLIT) // TC_BW
A = 10
SUB = TC_BW // 8

NEG_INF = float("-inf")
BIG = 1 << 30


def _merge(va, ia, vb, ib):
    take_b = (vb > va) | ((vb == va) & (ib < ia))
    return jnp.where(take_b, vb, va), jnp.where(take_b, ib, ia)


def _sc_body(y_hbm, oval_hbm, oidx_hbm,
             bufa, bufb, stage_v, stage_i, mrg_v, mrg_i, shv, shi,
             sema, semb):
    cid = lax.axis_index("c")
    sid = lax.axis_index("s")
    wid = sid * NC + cid
    base = jnp.minimum(wid * Q, Q_LAST0)
    base = pl.multiple_of(base, 8)
    lanes = lax.iota(jnp.int32, 16)

    bufs = (bufa, bufb)
    sems = (sema, semb)
    pend = [None, None]
    pend[0] = pltpu.make_async_copy(
        y_hbm.at[pl.ds(base, CKT), :], bufs[0], sems[0])
    pend[0].start()
    pend[1] = pltpu.make_async_copy(
        y_hbm.at[pl.ds(base + CKT, CKT), :], bufs[1], sems[1])
    pend[1].start()

    gv = [jnp.full((16,), NEG_INF, jnp.float32) for _ in range(8)]
    gi = [jnp.full((16,), BIG, jnp.int32) for _ in range(8)]

    for k in range(NCH):
        pend[k % 2].wait()
        buf = bufs[k % 2]
        t0 = base + k * CKT

        def step(t, carry, buf=buf, t0=t0):
            cv, ci = carry
            nv, ni = [], []
            col = t0 + t
            for g in range(8):
                v = buf[t, pl.ds(g * 16, 16)]
                m = v > cv[g]
                nv.append(jnp.where(m, v, cv[g]))
                ni.append(jnp.where(m, col, ci[g]))
            return tuple(nv), tuple(ni)

        gv, gi = lax.fori_loop(0, CKT, step, (tuple(gv), tuple(gi)))
        gv, gi = list(gv), list(gi)
        if k + 2 < NCH:
            pend[k % 2] = pltpu.make_async_copy(
                y_hbm.at[pl.ds(base + (k + 2) * CKT, CKT), :],
                bufs[k % 2], sems[k % 2])
            pend[k % 2].start()

    for g in range(8):
        stage_v[pl.ds(g * 16, 16)] = gv[g]
        stage_i[pl.ds(g * 16, 16)] = gi[g]
    sbase = pl.multiple_of(sid * 16, 16)
    for g in range(8):
        pltpu.sync_copy(stage_v.at[pl.ds(g * 16, 16)],
                        shv.at[pl.ds(g * 256 + sbase, 16)])
        pltpu.sync_copy(stage_i.at[pl.ds(g * 16, 16)],
                        shi.at[pl.ds(g * 256 + sbase, 16)])
    plsc.subcore_barrier()

    @pl.when(sid < 8)
    def _():
        gbase = pl.multiple_of(sid * 256, 16)
        pltpu.sync_copy(shv.at[pl.ds(gbase, 256)], mrg_v)
        pltpu.sync_copy(shi.at[pl.ds(gbase, 256)], mrg_i)
        bv = mrg_v[pl.ds(0, 16)]
        bi = mrg_i[pl.ds(0, 16)]
        for s in range(1, 16):
            bv, bi = _merge(bv, bi, mrg_v[pl.ds(s * 16, 16)],
                            mrg_i[pl.ds(s * 16, 16)])
        stage_v[pl.ds(0, 16)] = bv
        stage_i[pl.ds(0, 16)] = bi
        obase = pl.multiple_of(cid * R + sid * 16, 16)
        pltpu.sync_copy(stage_v.at[pl.ds(0, 16)],
                        oval_hbm.at[pl.ds(obase, 16)])
        pltpu.sync_copy(stage_i.at[pl.ds(0, 16)],
                        oidx_hbm.at[pl.ds(obase, 16)])


def _tc_body(y_ref, oval_ref, oidx_ref):
    i = pl.program_id(0)
    tb = (i + TC_BLK0) * TC_BW

    accv = [None] * A
    accj = [None] * A
    for j in range(SUB):
        a = j % A
        v = y_ref[pl.ds(j * 8, 8), :]
        if accv[a] is None:
            accv[a] = v
            accj[a] = jnp.full((8, 128), j, jnp.int32)
        else:
            m = v > accv[a]
            accv[a] = jnp.where(m, v, accv[a])
            accj[a] = jnp.where(m, jnp.int32(j), accj[a])

    bv, bj = accv[0], accj[0]
    for a in range(1, A):
        t = (accv[a] > bv) | ((accv[a] == bv) & (accj[a] < bj))
        bv = jnp.where(t, accv[a], bv)
        bj = jnp.where(t, accj[a], bj)

    sub = lax.broadcasted_iota(jnp.int32, (8, 128), 0)
    bc = bj * 8 + sub + tb
    vmax = jnp.max(bv, axis=0)
    cand = jnp.where(bv == vmax[None, :], bc, jnp.int32(BIG))
    cmin = jnp.min(cand, axis=0)

    @pl.when(i == 0)
    def _():
        oval_ref[0, 0, :] = vmax
        oidx_ref[0, 0, :] = cmin

    @pl.when(i > 0)
    def _():
        pv = oval_ref[0, 0, :]
        pi = oidx_ref[0, 0, :]
        t = (vmax > pv) | ((vmax == pv) & (cmin < pi))
        oval_ref[0, 0, :] = jnp.where(t, vmax, pv)
        oidx_ref[0, 0, :] = jnp.where(t, cmin, pi)


def kernel(x):
    y = jnp.transpose(x)

    mesh = plsc.VectorSubcoreMesh(core_axis_name="c", subcore_axis_name="s")
    sc_kern = pl.kernel(
        _sc_body,
        mesh=mesh,
        compiler_params=pltpu.CompilerParams(use_tc_tiling_on_sc=True),
        out_type=(
            jax.ShapeDtypeStruct((NC * R,), jnp.float32),
            jax.ShapeDtypeStruct((NC * R,), jnp.int32),
        ),
        scratch_types=[
            pltpu.VMEM((CKT, 128), jnp.float32),
            pltpu.VMEM((CKT, 128), jnp.float32),
            pltpu.VMEM((128,), jnp.float32),
            pltpu.VMEM((128,), jnp.int32),
            pltpu.VMEM((256,), jnp.float32),
            pltpu.VMEM((256,), jnp.int32),
            pltpu.VMEM_SHARED((2048,), jnp.float32),
            pltpu.VMEM_SHARED((2048,), jnp.int32),
            pltpu.SemaphoreType.DMA,
            pltpu.SemaphoreType.DMA,
        ],
    )
    sval, sidx = sc_kern(y)

    tval, tidx = pl.pallas_call(
        _tc_body,
        grid=(TC_NBLK,),
        in_specs=[pl.BlockSpec((TC_BW, 128), lambda i: (i + TC_BLK0, 0))],
        out_specs=[
            pl.BlockSpec((1, 1, 128), lambda i: (0, 0, 0)),
            pl.BlockSpec((1, 1, 128), lambda i: (0, 0, 0)),
        ],
        out_shape=[
            jax.ShapeDtypeStruct((1, 1, 128), jnp.float32),
            jax.ShapeDtypeStruct((1, 1, 128), jnp.int32),
        ],
    )(y)

    tv = tval.reshape(R)
    ti = tidx.reshape(R)
    sv = sval.reshape(NC, R)
    si = sidx.reshape(NC, R)

    v, i = sv[0], si[0]
    for vb, ib in ((sv[1], si[1]), (tv, ti)):
        t = (vb > v) | ((vb == v) & (ib < i))
        v = jnp.where(t, vb, v)
        i = jnp.where(t, ib, i)
    return i.astype(jnp.int64)


# speedup vs baseline: 1.5048x; 1.5048x over previous
import jax
import jax.numpy as jnp
from jax import lax
from jax.experimental import pallas as pl

R, C = 128, 100000
TC_BW = 4000
TC_NBLK = C // TC_BW
A = 10
SUB = TC_BW // 8

NEG_INF = float("-inf")
BIG = 1 << 30


def _tc_body(y_ref, oval_ref, oidx_ref):
    i = pl.program_id(0)
    tb = i * TC_BW

    accv = [None] * A
    accj = [None] * A
    for j in range(SUB):
        a = j % A
        v = y_ref[pl.ds(j * 8, 8), :]
        if accv[a] is None:
            accv[a] = v
            accj[a] = jnp.full((8, 128), j, jnp.int32)
        else:
            m = v > accv[a]
            accv[a] = jnp.where(m, v, accv[a])
            accj[a] = jnp.where(m, jnp.int32(j), accj[a])

    bv, bj = accv[0], accj[0]
    for a in range(1, A):
        t = (accv[a] > bv) | ((accv[a] == bv) & (accj[a] < bj))
        bv = jnp.where(t, accv[a], bv)
        bj = jnp.where(t, accj[a], bj)

    sub = lax.broadcasted_iota(jnp.int32, (8, 128), 0)
    bc = bj * 8 + sub + tb
    vmax = jnp.max(bv, axis=0)
    cand = jnp.where(bv == vmax[None, :], bc, jnp.int32(BIG))
    cmin = jnp.min(cand, axis=0)

    @pl.when(i == 0)
    def _():
        oval_ref[0, 0, :] = vmax
        oidx_ref[0, 0, :] = cmin

    @pl.when(i > 0)
    def _():
        pv = oval_ref[0, 0, :]
        pi = oidx_ref[0, 0, :]
        t = (vmax > pv) | ((vmax == pv) & (cmin < pi))
        oval_ref[0, 0, :] = jnp.where(t, vmax, pv)
        oidx_ref[0, 0, :] = jnp.where(t, cmin, pi)


def kernel(x):
    y = jnp.transpose(x)

    tval, tidx = pl.pallas_call(
        _tc_body,
        grid=(TC_NBLK,),
        in_specs=[pl.BlockSpec((TC_BW, 128), lambda i: (i, 0))],
        out_specs=[
            pl.BlockSpec((1, 1, 128), lambda i: (0, 0, 0)),
            pl.BlockSpec((1, 1, 128), lambda i: (0, 0, 0)),
        ],
        out_shape=[
            jax.ShapeDtypeStruct((1, 1, 128), jnp.float32),
            jax.ShapeDtypeStruct((1, 1, 128), jnp.int32),
        ],
    )(y)
    return tidx.reshape(R).astype(jnp.int64)


# speedup vs baseline: 2.1397x; 1.4219x over previous
import jax
import jax.numpy as jnp
from jax import lax
from jax.experimental import pallas as pl

R, C = 128, 100000
TC_BW = 10000
TC_NBLK = C // TC_BW
A = 10
SUB = TC_BW // 8

NEG_INF = float("-inf")
BIG = 1 << 30


def _tc_body(y_ref, oval_ref, oidx_ref):
    i = pl.program_id(0)
    tb = i * TC_BW

    accv = [None] * A
    accj = [None] * A
    for j in range(SUB):
        a = j % A
        v = y_ref[pl.ds(j * 8, 8), :]
        if accv[a] is None:
            accv[a] = v
            accj[a] = jnp.full((8, 128), j, jnp.int32)
        else:
            m = v > accv[a]
            accv[a] = jnp.where(m, v, accv[a])
            accj[a] = jnp.where(m, jnp.int32(j), accj[a])

    bv, bj = accv[0], accj[0]
    for a in range(1, A):
        t = (accv[a] > bv) | ((accv[a] == bv) & (accj[a] < bj))
        bv = jnp.where(t, accv[a], bv)
        bj = jnp.where(t, accj[a], bj)

    sub = lax.broadcasted_iota(jnp.int32, (8, 128), 0)
    bc = bj * 8 + sub + tb
    vmax = jnp.max(bv, axis=0)
    cand = jnp.where(bv == vmax[None, :], bc, jnp.int32(BIG))
    cmin = jnp.min(cand, axis=0)

    @pl.when(i == 0)
    def _():
        oval_ref[0, 0, :] = vmax
        oidx_ref[0, 0, :] = cmin

    @pl.when(i > 0)
    def _():
        pv = oval_ref[0, 0, :]
        pi = oidx_ref[0, 0, :]
        t = (vmax > pv) | ((vmax == pv) & (cmin < pi))
        oval_ref[0, 0, :] = jnp.where(t, vmax, pv)
        oidx_ref[0, 0, :] = jnp.where(t, cmin, pi)


def kernel(x):
    y = jnp.transpose(x)

    tval, tidx = pl.pallas_call(
        _tc_body,
        grid=(TC_NBLK,),
        in_specs=[pl.BlockSpec((TC_BW, 128), lambda i: (i, 0))],
        out_specs=[
            pl.BlockSpec((1, 1, 128), lambda i: (0, 0, 0)),
            pl.BlockSpec((1, 1, 128), lambda i: (0, 0, 0)),
        ],
        out_shape=[
            jax.ShapeDtypeStruct((1, 1, 128), jnp.float32),
            jax.ShapeDtypeStruct((1, 1, 128), jnp.int32),
        ],
    )(y)
    return tidx.reshape(R).astype(jnp.int64)
